# baseline (speedup 1.0000x reference)
import functools

import jax
import jax.numpy as jnp
from jax import lax
from jax.experimental import pallas as pl
from jax.experimental.pallas import tpu as pltpu
from jax.experimental.pallas import tpu_sc as plsc

NUM_CORES = 2
NUM_SUBCORES = 16
LANES = 16
NW = NUM_CORES * NUM_SUBCORES

B = 16384
D = 128
DSL = D // LANES
BPW = B // NW
CHUNK = 32
NCHUNK = BPW // CHUNK
NBUF = 3
GROUPS = CHUNK // LANES
TSTRIDE = LANES + 1
TTILE = LANES * TSTRIDE

_mesh = plsc.VectorSubcoreMesh(core_axis_name="c", subcore_axis_name="s")


@functools.partial(
    pl.kernel,
    out_type=jax.ShapeDtypeStruct((B,), jnp.float32),
    mesh=_mesh,
    compiler_params=pltpu.CompilerParams(needs_layout_passes=False),
    scratch_types=[
        pltpu.VMEM((NCHUNK, CHUNK), jnp.int32),
        pltpu.VMEM((NCHUNK, CHUNK), jnp.int32),
        pltpu.VMEM((NBUF * CHUNK, D), jnp.float32),
        pltpu.VMEM((NBUF * CHUNK, D), jnp.float32),
        pltpu.VMEM((BPW,), jnp.float32),
        pltpu.VMEM((BPW,), jnp.float32),
        pltpu.VMEM((2 * TTILE,), jnp.float32),
        pltpu.VMEM((BPW,), jnp.float32),
        pltpu.VMEM((1,), jnp.float32),
        pltpu.SemaphoreType.DMA,
        pltpu.SemaphoreType.DMA,
        pltpu.SemaphoreType.DMA,
    ],
)
def _mf_sc(user_hbm, item_hbm, utab_hbm, itab_hbm, ubias_hbm, ibias_hbm,
           off_hbm, out_hbm,
           uidx_v, iidx_v, urows, irows, ubias_v, ibias_v, ttile,
           out_v, off_v, sem, bsem, isem):
    wid = lax.axis_index("s") * NUM_CORES + lax.axis_index("c")
    base = wid * BPW

    icp0 = pltpu.async_copy(user_hbm.at[wid], uidx_v, isem)
    icp1 = pltpu.async_copy(item_hbm.at[wid], iidx_v, isem)
    pltpu.sync_copy(off_hbm, off_v)
    icp0.wait()
    icp1.wait()

    def issue(c, b):
        pltpu.async_copy(utab_hbm.at[uidx_v.at[c]],
                         urows.at[pl.ds(b, CHUNK)], sem)
        pltpu.async_copy(itab_hbm.at[iidx_v.at[c]],
                         irows.at[pl.ds(b, CHUNK)], sem)

    issue(0, 0)
    issue(1, CHUNK)
    bias_copies = []
    for c in range(NCHUNK):
        s = pl.ds(c * CHUNK, CHUNK)
        bias_copies.append(
            pltpu.async_copy(ubias_hbm.at[uidx_v.at[c]], ubias_v.at[s], bsem))
        bias_copies.append(
            pltpu.async_copy(ibias_hbm.at[iidx_v.at[c]], ibias_v.at[s], bsem))

    lane_iota = lax.iota(jnp.int32, LANES)
    tcol_iota = lane_iota * TSTRIDE

    def cbody(c, carry):
        @pl.when(c + 2 < NCHUNK)
        def _():
            issue(c + 2, lax.rem(c + 2, NBUF) * CHUNK)

        b = lax.rem(c, NBUF) * CHUNK
        pltpu.make_async_copy(utab_hbm.at[uidx_v.at[c]],
                              urows.at[pl.ds(b, CHUNK)], sem).wait()
        pltpu.make_async_copy(itab_hbm.at[iidx_v.at[c]],
                              irows.at[pl.ds(b, CHUNK)], sem).wait()

        def gbody(g):
            row0 = b + g * LANES
            tbase = (g % 2) * TTILE

            def rbody(r):
                urow = urows.at[row0 + r]
                irow = irows.at[row0 + r]
                parts = []
                for k in range(DSL):
                    s = pl.ds(k * LANES, LANES)
                    parts.append(urow[s] * irow[s])
                p = ((parts[0] + parts[1]) + (parts[2] + parts[3])) + \
                    ((parts[4] + parts[5]) + (parts[6] + parts[7]))
                plsc.store_scatter(ttile, [tcol_iota + (tbase + r)], p)

            plsc.parallel_loop(0, LANES, 1, unroll=2)(rbody)
            trows = [plsc.load_gather(ttile, [lane_iota + (tbase + j * TSTRIDE)])
                     for j in range(LANES)]
            while len(trows) > 1:
                trows = [trows[i] + trows[i + 1]
                         for i in range(0, len(trows), 2)]
            plsc.store_scatter(out_v, [lane_iota + (c * CHUNK + g * LANES)],
                               trows[0])

        plsc.parallel_loop(0, GROUPS, 1, unroll=1)(gbody)
        return carry

    lax.fori_loop(0, NCHUNK, cbody, 0)

    for cp in bias_copies:
        cp.wait()

    off = plsc.load_gather(off_v, [jnp.zeros((LANES,), jnp.int32)])

    def combine(i):
        idx = lane_iota + i * LANES
        x = (plsc.load_gather(out_v, [idx])
             + plsc.load_gather(ubias_v, [idx])
             + plsc.load_gather(ibias_v, [idx]) + off)
        plsc.store_scatter(out_v, [idx], 5.5 / (1.0 + jnp.exp(-x)))

    plsc.parallel_loop(0, BPW // LANES, 1, unroll=2)(combine)

    pltpu.sync_copy(out_v, out_hbm.at[pl.ds(base, BPW)])


@jax.jit
def kernel(user, item, user_emb_table, item_emb_table, user_bias, item_bias,
           offset):
    user = user.astype(jnp.int32).reshape(NW, NCHUNK, CHUNK)
    item = item.astype(jnp.int32).reshape(NW, NCHUNK, CHUNK)
    off = jnp.reshape(offset, (1,)).astype(jnp.float32)
    return _mf_sc(user, item, user_emb_table, item_emb_table,
                  user_bias, item_bias, off)


# speedup vs baseline: 1.0137x; 1.0137x over previous
import functools

import jax
import jax.numpy as jnp
from jax import lax
from jax.experimental import pallas as pl
from jax.experimental.pallas import tpu as pltpu
from jax.experimental.pallas import tpu_sc as plsc

NUM_CORES = 2
NUM_SUBCORES = 16
LANES = 16
NW = NUM_CORES * NUM_SUBCORES

B = 16384
D = 128
DSL = D // LANES
BPW = B // NW
CHUNK = 64
NCHUNK = BPW // CHUNK
NBUF = 4
GROUPS = CHUNK // LANES
TSTRIDE = LANES + 1
TTILE = LANES * TSTRIDE

_mesh = plsc.VectorSubcoreMesh(core_axis_name="c", subcore_axis_name="s")


@functools.partial(
    pl.kernel,
    out_type=jax.ShapeDtypeStruct((B,), jnp.float32),
    mesh=_mesh,
    compiler_params=pltpu.CompilerParams(needs_layout_passes=False),
    scratch_types=[
        pltpu.VMEM((NCHUNK, CHUNK), jnp.int32),
        pltpu.VMEM((NCHUNK, CHUNK), jnp.int32),
        pltpu.VMEM((NBUF * CHUNK, D), jnp.float32),
        pltpu.VMEM((NBUF * CHUNK, D), jnp.float32),
        pltpu.VMEM((BPW,), jnp.float32),
        pltpu.VMEM((BPW,), jnp.float32),
        pltpu.VMEM((2 * TTILE,), jnp.float32),
        pltpu.VMEM((BPW,), jnp.float32),
        pltpu.VMEM((1,), jnp.float32),
        pltpu.SemaphoreType.DMA,
        pltpu.SemaphoreType.DMA,
        pltpu.SemaphoreType.DMA,
    ],
)
def _mf_sc(user_hbm, item_hbm, utab_hbm, itab_hbm, ubias_hbm, ibias_hbm,
           off_hbm, out_hbm,
           uidx_v, iidx_v, urows, irows, ubias_v, ibias_v, ttile,
           out_v, off_v, sem, bsem, isem):
    wid = lax.axis_index("s") * NUM_CORES + lax.axis_index("c")
    base = wid * BPW

    icp0 = pltpu.async_copy(user_hbm.at[wid], uidx_v, isem)
    icp1 = pltpu.async_copy(item_hbm.at[wid], iidx_v, isem)
    pltpu.sync_copy(off_hbm, off_v)
    icp0.wait()
    icp1.wait()

    def issue(c, b):
        pltpu.async_copy(utab_hbm.at[uidx_v.at[c]],
                         urows.at[pl.ds(b, CHUNK)], sem)
        pltpu.async_copy(itab_hbm.at[iidx_v.at[c]],
                         irows.at[pl.ds(b, CHUNK)], sem)

    issue(0, 0)
    issue(1, CHUNK)
    issue(2, 2 * CHUNK)
    bias_copies = []
    for c in range(NCHUNK):
        s = pl.ds(c * CHUNK, CHUNK)
        bias_copies.append(
            pltpu.async_copy(ubias_hbm.at[uidx_v.at[c]], ubias_v.at[s], bsem))
        bias_copies.append(
            pltpu.async_copy(ibias_hbm.at[iidx_v.at[c]], ibias_v.at[s], bsem))

    lane_iota = lax.iota(jnp.int32, LANES)
    tcol_iota = lane_iota * TSTRIDE

    def cbody(c, carry):
        @pl.when(c + 3 < NCHUNK)
        def _():
            issue(c + 3, lax.rem(c + 3, NBUF) * CHUNK)

        b = lax.rem(c, NBUF) * CHUNK
        pltpu.make_async_copy(utab_hbm.at[uidx_v.at[c]],
                              urows.at[pl.ds(b, CHUNK)], sem).wait()
        pltpu.make_async_copy(itab_hbm.at[iidx_v.at[c]],
                              irows.at[pl.ds(b, CHUNK)], sem).wait()

        def gbody(g):
            row0 = b + g * LANES
            tbase = (g % 2) * TTILE

            def rbody(r):
                urow = urows.at[row0 + r]
                irow = irows.at[row0 + r]
                parts = []
                for k in range(DSL):
                    s = pl.ds(k * LANES, LANES)
                    parts.append(urow[s] * irow[s])
                p = ((parts[0] + parts[1]) + (parts[2] + parts[3])) + \
                    ((parts[4] + parts[5]) + (parts[6] + parts[7]))
                plsc.store_scatter(ttile, [tcol_iota + (tbase + r)], p)

            plsc.parallel_loop(0, LANES, 1, unroll=2)(rbody)
            trows = [plsc.load_gather(ttile, [lane_iota + (tbase + j * TSTRIDE)])
                     for j in range(LANES)]
            while len(trows) > 1:
                trows = [trows[i] + trows[i + 1]
                         for i in range(0, len(trows), 2)]
            plsc.store_scatter(out_v, [lane_iota + (c * CHUNK + g * LANES)],
                               trows[0])

        plsc.parallel_loop(0, GROUPS, 1, unroll=1)(gbody)
        return carry

    lax.fori_loop(0, NCHUNK, cbody, 0)

    for cp in bias_copies:
        cp.wait()

    off = plsc.load_gather(off_v, [jnp.zeros((LANES,), jnp.int32)])

    def combine(i):
        idx = lane_iota + i * LANES
        x = (plsc.load_gather(out_v, [idx])
             + plsc.load_gather(ubias_v, [idx])
             + plsc.load_gather(ibias_v, [idx]) + off)
        plsc.store_scatter(out_v, [idx], 5.5 / (1.0 + jnp.exp(-x)))

    plsc.parallel_loop(0, BPW // LANES, 1, unroll=2)(combine)

    pltpu.sync_copy(out_v, out_hbm.at[pl.ds(base, BPW)])


@jax.jit
def kernel(user, item, user_emb_table, item_emb_table, user_bias, item_bias,
           offset):
    user = user.astype(jnp.int32).reshape(NW, NCHUNK, CHUNK)
    item = item.astype(jnp.int32).reshape(NW, NCHUNK, CHUNK)
    off = jnp.reshape(offset, (1,)).astype(jnp.float32)
    return _mf_sc(user, item, user_emb_table, item_emb_table,
                  user_bias, item_bias, off)


# speedup vs baseline: 1.0230x; 1.0092x over previous
import functools

import jax
import jax.numpy as jnp
from jax import lax
from jax.experimental import pallas as pl
from jax.experimental.pallas import tpu as pltpu
from jax.experimental.pallas import tpu_sc as plsc

NUM_CORES = 2
NUM_SUBCORES = 16
LANES = 16
NW = NUM_CORES * NUM_SUBCORES

B = 16384
D = 128
DSL = D // LANES
BPW = B // NW
CHUNK = 64
NCHUNK = BPW // CHUNK
NBUF = 3
GROUPS = CHUNK // LANES
TSTRIDE = LANES + 1
TTILE = LANES * TSTRIDE

_mesh = plsc.VectorSubcoreMesh(core_axis_name="c", subcore_axis_name="s")


@functools.partial(
    pl.kernel,
    out_type=jax.ShapeDtypeStruct((B,), jnp.float32),
    mesh=_mesh,
    compiler_params=pltpu.CompilerParams(needs_layout_passes=False),
    scratch_types=[
        pltpu.VMEM((NCHUNK, CHUNK), jnp.int32),
        pltpu.VMEM((NCHUNK, CHUNK), jnp.int32),
        pltpu.VMEM((NBUF * CHUNK, D), jnp.float32),
        pltpu.VMEM((NBUF * CHUNK, D), jnp.float32),
        pltpu.VMEM((BPW,), jnp.float32),
        pltpu.VMEM((BPW,), jnp.float32),
        pltpu.VMEM((2 * TTILE,), jnp.float32),
        pltpu.VMEM((BPW,), jnp.float32),
        pltpu.VMEM((1,), jnp.float32),
        pltpu.SemaphoreType.DMA,
        pltpu.SemaphoreType.DMA,
        pltpu.SemaphoreType.DMA,
    ],
)
def _mf_sc(user_hbm, item_hbm, utab_hbm, itab_hbm, ubias_hbm, ibias_hbm,
           off_hbm, out_hbm,
           uidx_v, iidx_v, urows, irows, ubias_v, ibias_v, ttile,
           out_v, off_v, sem, bsem, isem):
    wid = lax.axis_index("s") * NUM_CORES + lax.axis_index("c")
    base = wid * BPW

    icp0 = pltpu.async_copy(user_hbm.at[wid], uidx_v, isem)
    icp1 = pltpu.async_copy(item_hbm.at[wid], iidx_v, isem)
    pltpu.sync_copy(off_hbm, off_v)
    icp0.wait()
    icp1.wait()

    def issue(c, b):
        pltpu.async_copy(utab_hbm.at[uidx_v.at[c]],
                         urows.at[pl.ds(b, CHUNK)], sem)
        pltpu.async_copy(itab_hbm.at[iidx_v.at[c]],
                         irows.at[pl.ds(b, CHUNK)], sem)

    issue(0, 0)
    issue(1, CHUNK)
    bias_copies = []
    for c in range(NCHUNK):
        s = pl.ds(c * CHUNK, CHUNK)
        bias_copies.append(
            pltpu.async_copy(ubias_hbm.at[uidx_v.at[c]], ubias_v.at[s], bsem))
        bias_copies.append(
            pltpu.async_copy(ibias_hbm.at[iidx_v.at[c]], ibias_v.at[s], bsem))

    lane_iota = lax.iota(jnp.int32, LANES)
    tcol_iota = lane_iota * TSTRIDE

    def cbody(c, carry):
        @pl.when(c + 2 < NCHUNK)
        def _():
            issue(c + 2, lax.rem(c + 2, NBUF) * CHUNK)

        b = lax.rem(c, NBUF) * CHUNK
        pltpu.make_async_copy(utab_hbm.at[uidx_v.at[c]],
                              urows.at[pl.ds(b, CHUNK)], sem).wait()
        pltpu.make_async_copy(itab_hbm.at[iidx_v.at[c]],
                              irows.at[pl.ds(b, CHUNK)], sem).wait()

        def gbody(g):
            row0 = b + g * LANES
            tbase = (g % 2) * TTILE

            def rbody(r):
                urow = urows.at[row0 + r]
                irow = irows.at[row0 + r]
                parts = []
                for k in range(DSL):
                    s = pl.ds(k * LANES, LANES)
                    parts.append(urow[s] * irow[s])
                p = ((parts[0] + parts[1]) + (parts[2] + parts[3])) + \
                    ((parts[4] + parts[5]) + (parts[6] + parts[7]))
                plsc.store_scatter(ttile, [tcol_iota + (tbase + r)], p)

            plsc.parallel_loop(0, LANES, 1, unroll=2)(rbody)
            trows = [plsc.load_gather(ttile, [lane_iota + (tbase + j * TSTRIDE)])
                     for j in range(LANES)]
            while len(trows) > 1:
                trows = [trows[i] + trows[i + 1]
                         for i in range(0, len(trows), 2)]
            plsc.store_scatter(out_v, [lane_iota + (c * CHUNK + g * LANES)],
                               trows[0])

        plsc.parallel_loop(0, GROUPS, 1, unroll=1)(gbody)
        return carry

    lax.fori_loop(0, NCHUNK, cbody, 0)

    for cp in bias_copies:
        cp.wait()

    off = plsc.load_gather(off_v, [jnp.zeros((LANES,), jnp.int32)])

    def combine(i):
        idx = lane_iota + i * LANES
        x = (plsc.load_gather(out_v, [idx])
             + plsc.load_gather(ubias_v, [idx])
             + plsc.load_gather(ibias_v, [idx]) + off)
        plsc.store_scatter(out_v, [idx], 5.5 / (1.0 + jnp.exp(-x)))

    plsc.parallel_loop(0, BPW // LANES, 1, unroll=2)(combine)

    pltpu.sync_copy(out_v, out_hbm.at[pl.ds(base, BPW)])


@jax.jit
def kernel(user, item, user_emb_table, item_emb_table, user_bias, item_bias,
           offset):
    user = user.astype(jnp.int32).reshape(NW, NCHUNK, CHUNK)
    item = item.astype(jnp.int32).reshape(NW, NCHUNK, CHUNK)
    off = jnp.reshape(offset, (1,)).astype(jnp.float32)
    return _mf_sc(user, item, user_emb_table, item_emb_table,
                  user_bias, item_bias, off)


# speedup vs baseline: 1.0492x; 1.0255x over previous
import functools

import jax
import jax.numpy as jnp
from jax import lax
from jax.experimental import pallas as pl
from jax.experimental.pallas import tpu as pltpu
from jax.experimental.pallas import tpu_sc as plsc

NUM_CORES = 2
NUM_SUBCORES = 16
LANES = 16
NW = NUM_CORES * NUM_SUBCORES

B = 16384
D = 128
DSL = D // LANES
BPW = B // NW
CHUNK = 64
NCHUNK = BPW // CHUNK
NBUF = 3
GROUPS = CHUNK // LANES
TSTRIDE = LANES + 1
TTILE = LANES * TSTRIDE

_mesh = plsc.VectorSubcoreMesh(core_axis_name="c", subcore_axis_name="s")


@functools.partial(
    pl.kernel,
    out_type=jax.ShapeDtypeStruct((B,), jnp.float32),
    mesh=_mesh,
    compiler_params=pltpu.CompilerParams(needs_layout_passes=False),
    scratch_types=[
        pltpu.VMEM((BPW,), jnp.int32),
        pltpu.VMEM((BPW,), jnp.int32),
        pltpu.VMEM((NBUF * CHUNK, D), jnp.float32),
        pltpu.VMEM((NBUF * CHUNK, D), jnp.float32),
        pltpu.VMEM((BPW,), jnp.float32),
        pltpu.VMEM((BPW,), jnp.float32),
        pltpu.VMEM((2 * TTILE,), jnp.float32),
        pltpu.VMEM((BPW,), jnp.float32),
        pltpu.VMEM((1,), jnp.float32),
        pltpu.SemaphoreType.DMA,
        pltpu.SemaphoreType.DMA,
        pltpu.SemaphoreType.DMA,
    ],
)
def _mf_sc(user_hbm, item_hbm, utab_hbm, itab_hbm, ubias_hbm, ibias_hbm,
           off_hbm, out_hbm,
           uidx_v, iidx_v, urows, irows, ubias_v, ibias_v, ttile,
           out_v, off_v, sem, bsem, isem):
    wid = lax.axis_index("s") * NUM_CORES + lax.axis_index("c")
    base = wid * BPW

    icp0 = pltpu.async_copy(user_hbm.at[pl.ds(base, BPW)], uidx_v, isem)
    icp1 = pltpu.async_copy(item_hbm.at[pl.ds(base, BPW)], iidx_v, isem)
    pltpu.sync_copy(off_hbm, off_v)
    icp0.wait()
    icp1.wait()

    def issue(c, b):
        s = pl.ds(c * CHUNK, CHUNK)
        pltpu.async_copy(utab_hbm.at[uidx_v.at[s]],
                         urows.at[pl.ds(b, CHUNK)], sem)
        pltpu.async_copy(itab_hbm.at[iidx_v.at[s]],
                         irows.at[pl.ds(b, CHUNK)], sem)

    issue(0, 0)
    issue(1, CHUNK)
    bias_copies = []
    for c in range(BPW // 128):
        s = pl.ds(c * 128, 128)
        bias_copies.append(
            pltpu.async_copy(ubias_hbm.at[uidx_v.at[s]], ubias_v.at[s], bsem))
        bias_copies.append(
            pltpu.async_copy(ibias_hbm.at[iidx_v.at[s]], ibias_v.at[s], bsem))

    lane_iota = lax.iota(jnp.int32, LANES)
    tcol_iota = lane_iota * TSTRIDE

    def cbody(c, carry):
        @pl.when(c + 2 < NCHUNK)
        def _():
            issue(c + 2, lax.rem(c + 2, NBUF) * CHUNK)

        b = lax.rem(c, NBUF) * CHUNK
        s = pl.ds(c * CHUNK, CHUNK)
        pltpu.make_async_copy(utab_hbm.at[uidx_v.at[s]],
                              urows.at[pl.ds(b, CHUNK)], sem).wait()
        pltpu.make_async_copy(itab_hbm.at[iidx_v.at[s]],
                              irows.at[pl.ds(b, CHUNK)], sem).wait()

        def gbody(g):
            row0 = b + g * LANES
            tbase = (g % 2) * TTILE

            def rbody(r):
                urow = urows.at[row0 + r]
                irow = irows.at[row0 + r]
                parts = []
                for k in range(DSL):
                    s = pl.ds(k * LANES, LANES)
                    parts.append(urow[s] * irow[s])
                p = ((parts[0] + parts[1]) + (parts[2] + parts[3])) + \
                    ((parts[4] + parts[5]) + (parts[6] + parts[7]))
                plsc.store_scatter(ttile, [tcol_iota + (tbase + r)], p)

            plsc.parallel_loop(0, LANES, 1, unroll=2)(rbody)
            trows = [plsc.load_gather(ttile, [lane_iota + (tbase + j * TSTRIDE)])
                     for j in range(LANES)]
            while len(trows) > 1:
                trows = [trows[i] + trows[i + 1]
                         for i in range(0, len(trows), 2)]
            plsc.store_scatter(out_v, [lane_iota + (c * CHUNK + g * LANES)],
                               trows[0])

        plsc.parallel_loop(0, GROUPS, 1, unroll=1)(gbody)
        return carry

    lax.fori_loop(0, NCHUNK, cbody, 0)

    for cp in bias_copies:
        cp.wait()

    off = plsc.load_gather(off_v, [jnp.zeros((LANES,), jnp.int32)])

    def combine(i):
        idx = lane_iota + i * LANES
        x = (plsc.load_gather(out_v, [idx])
             + plsc.load_gather(ubias_v, [idx])
             + plsc.load_gather(ibias_v, [idx]) + off)
        plsc.store_scatter(out_v, [idx], 5.5 / (1.0 + jnp.exp(-x)))

    plsc.parallel_loop(0, BPW // LANES, 1, unroll=2)(combine)

    pltpu.sync_copy(out_v, out_hbm.at[pl.ds(base, BPW)])


@jax.jit
def kernel(user, item, user_emb_table, item_emb_table, user_bias, item_bias,
           offset):
    user = user.astype(jnp.int32)
    item = item.astype(jnp.int32)
    off = jnp.reshape(offset, (1,)).astype(jnp.float32)
    return _mf_sc(user, item, user_emb_table, item_emb_table,
                  user_bias, item_bias, off)


# speedup vs baseline: 1.0560x; 1.0065x over previous
import functools

import jax
import jax.numpy as jnp
from jax import lax
from jax.experimental import pallas as pl
from jax.experimental.pallas import tpu as pltpu
from jax.experimental.pallas import tpu_sc as plsc

NUM_CORES = 2
NUM_SUBCORES = 16
LANES = 16
NW = NUM_CORES * NUM_SUBCORES

B = 16384
D = 128
DSL = D // LANES
BPW = B // NW
CHUNK = 64
NCHUNK = BPW // CHUNK
NBUF = 3
GROUPS = CHUNK // LANES
TSTRIDE = LANES + 1
TTILE = LANES * TSTRIDE

_mesh = plsc.VectorSubcoreMesh(core_axis_name="c", subcore_axis_name="s")


@functools.partial(
    pl.kernel,
    out_type=jax.ShapeDtypeStruct((B,), jnp.float32),
    mesh=_mesh,
    compiler_params=pltpu.CompilerParams(needs_layout_passes=False),
    scratch_types=[
        pltpu.VMEM((BPW,), jnp.int32),
        pltpu.VMEM((BPW,), jnp.int32),
        pltpu.VMEM((NBUF * CHUNK, D), jnp.float32),
        pltpu.VMEM((NBUF * CHUNK, D), jnp.float32),
        pltpu.VMEM((BPW,), jnp.float32),
        pltpu.VMEM((BPW,), jnp.float32),
        pltpu.VMEM((2 * TTILE,), jnp.float32),
        pltpu.VMEM((BPW,), jnp.float32),
        pltpu.VMEM((1,), jnp.float32),
        pltpu.SemaphoreType.DMA,
        pltpu.SemaphoreType.DMA,
        pltpu.SemaphoreType.DMA,
    ],
)
def _mf_sc(user_hbm, item_hbm, utab_hbm, itab_hbm, ubias_hbm, ibias_hbm,
           off_hbm, out_hbm,
           uidx_v, iidx_v, urows, irows, ubias_v, ibias_v, ttile,
           out_v, off_v, sem, bsem, isem):
    wid = lax.axis_index("s") * NUM_CORES + lax.axis_index("c")
    base = wid * BPW

    icp0 = pltpu.async_copy(user_hbm.at[pl.ds(base, BPW)], uidx_v, isem)
    icp1 = pltpu.async_copy(item_hbm.at[pl.ds(base, BPW)], iidx_v, isem)
    pltpu.sync_copy(off_hbm, off_v)
    icp0.wait()
    icp1.wait()

    def issue(c, b):
        s = pl.ds(c * CHUNK, CHUNK)
        pltpu.async_copy(utab_hbm.at[uidx_v.at[s]],
                         urows.at[pl.ds(b, CHUNK)], sem)
        pltpu.async_copy(itab_hbm.at[iidx_v.at[s]],
                         irows.at[pl.ds(b, CHUNK)], sem)

    issue(0, 0)
    issue(1, CHUNK)
    bias_copies = []
    for c in range(BPW // 128):
        s = pl.ds(c * 128, 128)
        bias_copies.append(
            pltpu.async_copy(ubias_hbm.at[uidx_v.at[s]], ubias_v.at[s], bsem))
        bias_copies.append(
            pltpu.async_copy(ibias_hbm.at[iidx_v.at[s]], ibias_v.at[s], bsem))

    lane_iota = lax.iota(jnp.int32, LANES)
    tcol_iota = lane_iota * TSTRIDE

    def cbody(c, carry):
        @pl.when(c + 2 < NCHUNK)
        def _():
            issue(c + 2, lax.rem(c + 2, NBUF) * CHUNK)

        b = lax.rem(c, NBUF) * CHUNK
        s = pl.ds(c * CHUNK, CHUNK)
        pltpu.make_async_copy(utab_hbm.at[uidx_v.at[s]],
                              urows.at[pl.ds(b, CHUNK)], sem).wait()
        pltpu.make_async_copy(itab_hbm.at[iidx_v.at[s]],
                              irows.at[pl.ds(b, CHUNK)], sem).wait()

        def gbody(g):
            row0 = b + g * LANES
            tbase = (g % 2) * TTILE

            def rbody(r):
                urow = urows.at[row0 + r]
                irow = irows.at[row0 + r]
                parts = []
                for k in range(DSL):
                    s = pl.ds(k * LANES, LANES)
                    parts.append(urow[s] * irow[s])
                p = ((parts[0] + parts[1]) + (parts[2] + parts[3])) + \
                    ((parts[4] + parts[5]) + (parts[6] + parts[7]))
                plsc.store_scatter(ttile, [tcol_iota + (tbase + r)], p)

            plsc.parallel_loop(0, LANES, 1, unroll=2)(rbody)
            trows = [plsc.load_gather(ttile, [lane_iota + (tbase + j * TSTRIDE)])
                     for j in range(LANES)]
            while len(trows) > 1:
                trows = [trows[i] + trows[i + 1]
                         for i in range(0, len(trows), 2)]
            plsc.store_scatter(out_v, [lane_iota + (c * CHUNK + g * LANES)],
                               trows[0])

        plsc.parallel_loop(0, GROUPS, 1, unroll=2)(gbody)
        return carry

    lax.fori_loop(0, NCHUNK, cbody, 0)

    for cp in bias_copies:
        cp.wait()

    off = plsc.load_gather(off_v, [jnp.zeros((LANES,), jnp.int32)])

    def combine(i):
        idx = lane_iota + i * LANES
        x = (plsc.load_gather(out_v, [idx])
             + plsc.load_gather(ubias_v, [idx])
             + plsc.load_gather(ibias_v, [idx]) + off)
        plsc.store_scatter(out_v, [idx], 5.5 / (1.0 + jnp.exp(-x)))

    plsc.parallel_loop(0, BPW // LANES, 1, unroll=2)(combine)

    pltpu.sync_copy(out_v, out_hbm.at[pl.ds(base, BPW)])


@jax.jit
def kernel(user, item, user_emb_table, item_emb_table, user_bias, item_bias,
           offset):
    user = user.astype(jnp.int32)
    item = item.astype(jnp.int32)
    off = jnp.reshape(offset, (1,)).astype(jnp.float32)
    return _mf_sc(user, item, user_emb_table, item_emb_table,
                  user_bias, item_bias, off)
